# baseline (speedup 1.0000x reference)
import jax
import jax.numpy as jnp
from jax.experimental import pallas as pl


def _mlp2(x, p):
    h = x @ p['w1'] + p['b1']
    h = jax.nn.leaky_relu(h)
    return h @ p['w2'] + p['b2']


def _ln(x, g, b):
    m = jnp.mean(x, axis=-1, keepdims=True)
    v = jnp.var(x, axis=-1, keepdims=True)
    return (x - m) / jnp.sqrt(v + 1e-5) * g + b


def _head_body(h_ref, w1_ref, b1_ref, w2_ref, b2_ref, o_ref):
    h = h_ref[...]
    a = jax.nn.leaky_relu(h @ w1_ref[...] + b1_ref[...][None, :])
    o_ref[...] = a @ w2_ref[...] + b2_ref[...][None, :]


def _head(h, p1, p2):
    n, d = h.shape
    dh = p1['w'].shape[1]
    do = p2['w'].shape[1]
    blk = 2000
    grid = (n // blk,)
    return pl.pallas_call(
        _head_body,
        grid=grid,
        in_specs=[
            pl.BlockSpec((blk, d), lambda i: (i, 0)),
            pl.BlockSpec((d, dh), lambda i: (0, 0)),
            pl.BlockSpec((dh,), lambda i: (0,)),
            pl.BlockSpec((dh, do), lambda i: (0, 0)),
            pl.BlockSpec((do,), lambda i: (0,)),
        ],
        out_specs=pl.BlockSpec((blk, do), lambda i: (i, 0)),
        out_shape=jax.ShapeDtypeStruct((n, do), h.dtype),
    )(h, p1['w'], p1['b'], p2['w'], p2['b'])


def kernel(x_node, x_net, edge_index_node_to_net, edge_weight_node_to_net,
           edge_type_node_to_net, edge_index_net_to_node,
           edge_weight_net_to_node, params):
    ei1 = edge_index_node_to_net
    ei2 = edge_index_net_to_node
    ew1 = edge_weight_node_to_net
    ew2 = edge_weight_net_to_node
    n_net = x_net.shape[0]
    n_node = x_node.shape[0]

    h_inst = _mlp2(x_node, params['node_enc'])
    h_net = _mlp2(x_net, params['net_enc'])
    for lp in params['layers']:
        msg = h_inst[ei1[0]] * ew1[:, None]
        agg_net = jax.ops.segment_sum(msg, ei1[1], num_segments=n_net)
        h_net = h_net + (agg_net @ lp['w_n2n'] + lp['b_n2n'])
        msg2 = h_net[ei2[0]] * ew2[:, None]
        agg_node = jax.ops.segment_sum(msg2, ei2[1], num_segments=n_node)
        h_inst = h_inst + (agg_node @ lp['w_net2n'] + lp['b_net2n'])
        h_inst = jax.nn.leaky_relu(_ln(h_inst, lp['ln_g'], lp['ln_b']))
        h_net = jax.nn.leaky_relu(_ln(h_net, lp['ln_g'], lp['ln_b']))
    out_inst = _head(h_inst, params['fc1_node'], params['fc2_node'])
    out_net = _head(h_net, params['fc1_net'], params['fc2_net'])
    return out_inst, out_net


# speedup vs baseline: 5.8422x; 5.8422x over previous
import functools

import jax
import jax.numpy as jnp
from jax import lax
from jax.experimental import pallas as pl
from jax.experimental.pallas import tpu as pltpu
from jax.experimental.pallas import tpu_sc as plsc

N = 10000
D = 256
DH = D // 2
E = 320000
NS = 16
NC = 2
ES = E // NS
CHUNK = 2000
NCHUNK = ES // CHUNK
K = 40
NWIN = CHUNK // K
ACC_ROWS = 10240


def _sc_body(h3, src, dst, w, out3,
             stg_src, stg_dst, stg_w, r0, r1, m0, m1, acc, g0, g1, s0, s1):
    c = lax.axis_index("c")
    s = lax.axis_index("s")
    zf = jnp.zeros((16,), jnp.float32)

    def _zrow(e, _):
        for k in range(DH // 16):
            m0[e, pl.ds(16 * k, 16)] = zf
        return 0
    lax.fori_loop(0, K, _zrow, 0)
    zslab = ACC_ROWS // NS
    for k in range(zslab // K):
        pltpu.sync_copy(m0, acc.at[pl.ds(s * zslab + k * K, K)])

    plsc.subcore_barrier()

    bufs = ((r0, m0, g0, s0), (r1, m1, g1, s1))
    ebase = s * ES

    def _gather(wi, ri, gi):
        return pltpu.make_async_copy(
            h3.at[c].at[stg_src.at[pl.ds(wi * K, K)]], ri, gi)

    def _scatter(wi, mi, si):
        return pltpu.make_async_copy(
            mi, acc.at[stg_dst.at[pl.ds(wi * K, K)]], si)

    def _mul(base_e, ri, mi):
        def _edge(e, _):
            wv = plsc.load_gather(stg_w, [jnp.broadcast_to(base_e + e, (16,))])
            for k in range(DH // 16):
                mi[e, pl.ds(16 * k, 16)] = ri[e, pl.ds(16 * k, 16)] * wv
            return 0
        lax.fori_loop(0, K, _edge, 0)

    def _chunk(ci, _):
        st = ebase + ci * CHUNK
        pltpu.sync_copy(src.at[pl.ds(st, CHUNK)], stg_src)
        pltpu.sync_copy(dst.at[pl.ds(st, CHUNK)], stg_dst)
        pltpu.sync_copy(w.at[pl.ds(st, CHUNK)], stg_w)

        for i in range(2):
            _gather(i, bufs[i][0], bufs[i][2]).start()

        def _pair(j2, _):
            for i in range(2):
                ri, mi, gi, si = bufs[i]
                wi = 2 * j2 + i
                _gather(wi, ri, gi).wait()

                @pl.when(wi >= 2)
                def _():
                    _scatter(wi - 2, mi, si).wait()

                _mul(wi * K, ri, mi)
                pltpu.async_copy(
                    mi, acc.at[stg_dst.at[pl.ds(wi * K, K)]], si, add=True)

                @pl.when(wi + 2 < NWIN)
                def _():
                    _gather(wi + 2, ri, gi).start()
            return 0

        lax.fori_loop(0, NWIN // 2, _pair, 0)
        for i in range(2):
            _scatter(NWIN - 2 + i, bufs[i][1], bufs[i][3]).wait()
        return 0

    lax.fori_loop(0, NCHUNK, _chunk, 0)

    plsc.subcore_barrier()

    wb = N // NS
    for k in range(wb // K):
        pltpu.sync_copy(acc.at[pl.ds(s * wb + k * K, K)],
                        out3.at[c, pl.ds(s * wb + k * K, K)])
    rem = wb - (wb // K) * K
    pltpu.sync_copy(acc.at[pl.ds(s * wb + wb - rem, rem)],
                    out3.at[c, pl.ds(s * wb + wb - rem, rem)])


_segsum = functools.partial(
    pl.kernel,
    out_type=jax.ShapeDtypeStruct((NC, N, DH), jnp.float32),
    mesh=plsc.VectorSubcoreMesh(
        core_axis_name="c", subcore_axis_name="s",
        num_cores=NC, num_subcores=NS),
    compiler_params=pltpu.CompilerParams(
        use_tc_tiling_on_sc=False, needs_layout_passes=False),
    scratch_types=[
        pltpu.VMEM((CHUNK,), jnp.int32),
        pltpu.VMEM((CHUNK,), jnp.int32),
        pltpu.VMEM((CHUNK,), jnp.float32),
        pltpu.VMEM((K, DH), jnp.float32),
        pltpu.VMEM((K, DH), jnp.float32),
        pltpu.VMEM((K, DH), jnp.float32),
        pltpu.VMEM((K, DH), jnp.float32),
        pltpu.VMEM_SHARED((ACC_ROWS, DH), jnp.float32),
        pltpu.SemaphoreType.DMA,
        pltpu.SemaphoreType.DMA,
        pltpu.SemaphoreType.DMA,
        pltpu.SemaphoreType.DMA,
    ],
)(_sc_body)


def segsum(h3, src, dst, w):
    return _segsum(h3, src, dst, w)


_BLK = 2000
_GRID = (N // _BLK,)


def _split_spec():
    return pl.BlockSpec((NC, _BLK, DH), lambda i: (0, i, 0))


def _row_spec(d):
    return pl.BlockSpec((_BLK, d), lambda i: (i, 0))


def _full_spec(a, b):
    return pl.BlockSpec((a, b), lambda i: (0, 0))


def _vec_spec(d):
    return pl.BlockSpec((d,), lambda i: (0,))


def _split_out(d=DH):
    return jax.ShapeDtypeStruct((NC, N, d), jnp.float32)


def _leaky(x):
    return jnp.where(x >= 0, x, 0.01 * x)


def _join(ref):
    return jnp.concatenate([ref[0], ref[1]], axis=-1)


def _store_split(o_ref, y):
    o_ref[0] = y[:, :DH]
    o_ref[1] = y[:, DH:]


def _enc_body(x_ref, w1_ref, b1_ref, w2_ref, b2_ref, o_ref):
    h = _leaky(jnp.dot(x_ref[...], w1_ref[...],
                       preferred_element_type=jnp.float32) + b1_ref[...][None, :])
    _store_split(o_ref, jnp.dot(h, w2_ref[...],
                                preferred_element_type=jnp.float32)
                 + b2_ref[...][None, :])


def _enc(x, p):
    di, dh = p['w1'].shape
    return pl.pallas_call(
        _enc_body, grid=_GRID,
        in_specs=[_row_spec(di), _full_spec(di, dh), _vec_spec(dh),
                  _full_spec(dh, D), _vec_spec(D)],
        out_specs=_split_spec(),
        out_shape=_split_out(),
    )(x, p['w1'], p['b1'], p['w2'], p['b2'])


def _addmm_body(h_ref, a_ref, w_ref, b_ref, o_ref):
    _store_split(o_ref, _join(h_ref) + jnp.dot(
        _join(a_ref), w_ref[...],
        preferred_element_type=jnp.float32) + b_ref[...][None, :])


def _addmm(h, a, w, b):
    return pl.pallas_call(
        _addmm_body, grid=_GRID,
        in_specs=[_split_spec(), _split_spec(), _full_spec(D, D),
                  _vec_spec(D)],
        out_specs=_split_spec(),
        out_shape=_split_out(),
    )(h, a, w, b)


def _ln_act(x, g, b):
    m = jnp.mean(x, axis=-1, keepdims=True)
    v = jnp.mean((x - m) * (x - m), axis=-1, keepdims=True)
    return _leaky((x - m) / jnp.sqrt(v + 1e-5) * g[None, :] + b[None, :])


def _addmm_ln_body(h_ref, a_ref, w_ref, b_ref, g_ref, bn_ref, o_ref):
    t = _join(h_ref) + jnp.dot(_join(a_ref), w_ref[...],
                               preferred_element_type=jnp.float32) + b_ref[...][None, :]
    _store_split(o_ref, _ln_act(t, g_ref[...], bn_ref[...]))


def _addmm_ln(h, a, w, b, g, bn):
    return pl.pallas_call(
        _addmm_ln_body, grid=_GRID,
        in_specs=[_split_spec(), _split_spec(), _full_spec(D, D),
                  _vec_spec(D), _vec_spec(D), _vec_spec(D)],
        out_specs=_split_spec(),
        out_shape=_split_out(),
    )(h, a, w, b, g, bn)


def _lnact_body(x_ref, g_ref, bn_ref, o_ref):
    _store_split(o_ref, _ln_act(_join(x_ref), g_ref[...], bn_ref[...]))


def _lnact(x, g, bn):
    return pl.pallas_call(
        _lnact_body, grid=_GRID,
        in_specs=[_split_spec(), _vec_spec(D), _vec_spec(D)],
        out_specs=_split_spec(),
        out_shape=_split_out(),
    )(x, g, bn)


def _head_body(h_ref, w1_ref, b1_ref, w2_ref, b2_ref, o_ref):
    a = _leaky(jnp.dot(_join(h_ref), w1_ref[...],
                       preferred_element_type=jnp.float32) + b1_ref[...][None, :])
    o_ref[...] = jnp.dot(a, w2_ref[...],
                         preferred_element_type=jnp.float32) + b2_ref[...][None, :]


def _head(h, p1, p2):
    dh = p1['w'].shape[1]
    do = p2['w'].shape[1]
    return pl.pallas_call(
        _head_body, grid=_GRID,
        in_specs=[_split_spec(), _full_spec(D, dh), _vec_spec(dh),
                  _full_spec(dh, do), _vec_spec(do)],
        out_specs=pl.BlockSpec((_BLK, do), lambda i: (i, 0)),
        out_shape=jax.ShapeDtypeStruct((N, do), jnp.float32),
    )(h, p1['w'], p1['b'], p2['w'], p2['b'])


def kernel(x_node, x_net, edge_index_node_to_net, edge_weight_node_to_net,
           edge_type_node_to_net, edge_index_net_to_node,
           edge_weight_net_to_node, params):
    src1 = edge_index_node_to_net[0]
    dst1 = edge_index_node_to_net[1]
    src2 = edge_index_net_to_node[0]
    dst2 = edge_index_net_to_node[1]
    ew1 = edge_weight_node_to_net
    ew2 = edge_weight_net_to_node

    h_inst = _enc(x_node, params['node_enc'])
    h_net = _enc(x_net, params['net_enc'])
    for lp in params['layers']:
        agg_net = segsum(h_inst, src1, dst1, ew1)
        tmp_net = _addmm(h_net, agg_net, lp['w_n2n'], lp['b_n2n'])
        agg_node = segsum(tmp_net, src2, dst2, ew2)
        h_inst = _addmm_ln(h_inst, agg_node, lp['w_net2n'], lp['b_net2n'],
                           lp['ln_g'], lp['ln_b'])
        h_net = _lnact(tmp_net, lp['ln_g'], lp['ln_b'])
    out_inst = _head(h_inst, params['fc1_node'], params['fc2_node'])
    out_net = _head(h_net, params['fc1_net'], params['fc2_net'])
    return out_inst, out_net


# speedup vs baseline: 6.3922x; 1.0942x over previous
import functools

import jax
import jax.numpy as jnp
from jax import lax
from jax.experimental import pallas as pl
from jax.experimental.pallas import tpu as pltpu
from jax.experimental.pallas import tpu_sc as plsc

N = 10000
D = 256
DH = D // 2
E = 320000
NS = 16
NC = 2
ES = E // NS
CHUNK = 2000
NCHUNK = ES // CHUNK
K = 40
NWIN = CHUNK // K
ACC_ROWS = 10240


def _sc_body(h3, src, dst, w, out3,
             stg_src, stg_dst, stg_w, r0, r1, m0, m1, acc, g0, g1, s0, s1):
    c = lax.axis_index("c")
    s = lax.axis_index("s")
    zf = jnp.zeros((16,), jnp.float32)

    def _zrow(e, _):
        for k in range(DH // 16):
            m0[e, pl.ds(16 * k, 16)] = zf
        return 0
    lax.fori_loop(0, K, _zrow, 0)
    zslab = ACC_ROWS // NS
    for k in range(zslab // K):
        pltpu.sync_copy(m0, acc.at[pl.ds(s * zslab + k * K, K)])

    plsc.subcore_barrier()

    bufs = ((r0, m0, g0, s0), (r1, m1, g1, s1))
    ebase = s * ES

    def _gather(wi, ri, gi):
        return pltpu.make_async_copy(
            h3.at[c].at[stg_src.at[pl.ds(wi * K, K)]], ri, gi)

    def _scatter(wi, mi, si):
        return pltpu.make_async_copy(
            mi, acc.at[stg_dst.at[pl.ds(wi * K, K)]], si)

    def _mul(base_e, ri, mi):
        @plsc.parallel_loop(0, K, unroll=4)
        def _edge(e):
            wv = plsc.load_gather(stg_w, [jnp.broadcast_to(base_e + e, (16,))])
            for k in range(DH // 16):
                mi[e, pl.ds(16 * k, 16)] = ri[e, pl.ds(16 * k, 16)] * wv

    def _chunk(ci, _):
        st = ebase + ci * CHUNK
        pltpu.sync_copy(src.at[pl.ds(st, CHUNK)], stg_src)
        pltpu.sync_copy(dst.at[pl.ds(st, CHUNK)], stg_dst)
        pltpu.sync_copy(w.at[pl.ds(st, CHUNK)], stg_w)

        for i in range(2):
            _gather(i, bufs[i][0], bufs[i][2]).start()

        def _pair(j2, _):
            for i in range(2):
                ri, mi, gi, si = bufs[i]
                wi = 2 * j2 + i
                _gather(wi, ri, gi).wait()

                @pl.when(wi >= 2)
                def _():
                    _scatter(wi - 2, mi, si).wait()

                _mul(wi * K, ri, mi)
                pltpu.async_copy(
                    mi, acc.at[stg_dst.at[pl.ds(wi * K, K)]], si, add=True)

                @pl.when(wi + 2 < NWIN)
                def _():
                    _gather(wi + 2, ri, gi).start()
            return 0

        lax.fori_loop(0, NWIN // 2, _pair, 0)
        for i in range(2):
            _scatter(NWIN - 2 + i, bufs[i][1], bufs[i][3]).wait()
        return 0

    lax.fori_loop(0, NCHUNK, _chunk, 0)

    plsc.subcore_barrier()

    wb = N // NS
    for k in range(wb // K):
        pltpu.sync_copy(acc.at[pl.ds(s * wb + k * K, K)],
                        out3.at[c, pl.ds(s * wb + k * K, K)])
    rem = wb - (wb // K) * K
    pltpu.sync_copy(acc.at[pl.ds(s * wb + wb - rem, rem)],
                    out3.at[c, pl.ds(s * wb + wb - rem, rem)])


_segsum = functools.partial(
    pl.kernel,
    out_type=jax.ShapeDtypeStruct((NC, N, DH), jnp.float32),
    mesh=plsc.VectorSubcoreMesh(
        core_axis_name="c", subcore_axis_name="s",
        num_cores=NC, num_subcores=NS),
    compiler_params=pltpu.CompilerParams(
        use_tc_tiling_on_sc=False, needs_layout_passes=False),
    scratch_types=[
        pltpu.VMEM((CHUNK,), jnp.int32),
        pltpu.VMEM((CHUNK,), jnp.int32),
        pltpu.VMEM((CHUNK,), jnp.float32),
        pltpu.VMEM((K, DH), jnp.float32),
        pltpu.VMEM((K, DH), jnp.float32),
        pltpu.VMEM((K, DH), jnp.float32),
        pltpu.VMEM((K, DH), jnp.float32),
        pltpu.VMEM_SHARED((ACC_ROWS, DH), jnp.float32),
        pltpu.SemaphoreType.DMA,
        pltpu.SemaphoreType.DMA,
        pltpu.SemaphoreType.DMA,
        pltpu.SemaphoreType.DMA,
    ],
)(_sc_body)


def segsum(h3, src, dst, w):
    return _segsum(h3, src, dst, w)


_BLK = 2000
_GRID = (N // _BLK,)


def _split_spec():
    return pl.BlockSpec((NC, _BLK, DH), lambda i: (0, i, 0))


def _row_spec(d):
    return pl.BlockSpec((_BLK, d), lambda i: (i, 0))


def _full_spec(a, b):
    return pl.BlockSpec((a, b), lambda i: (0, 0))


def _vec_spec(d):
    return pl.BlockSpec((d,), lambda i: (0,))


def _split_out(d=DH):
    return jax.ShapeDtypeStruct((NC, N, d), jnp.float32)


def _leaky(x):
    return jnp.where(x >= 0, x, 0.01 * x)


def _join(ref):
    return jnp.concatenate([ref[0], ref[1]], axis=-1)


def _store_split(o_ref, y):
    o_ref[0] = y[:, :DH]
    o_ref[1] = y[:, DH:]


def _enc_body(x_ref, w1_ref, b1_ref, w2_ref, b2_ref, o_ref):
    h = _leaky(jnp.dot(x_ref[...], w1_ref[...],
                       preferred_element_type=jnp.float32) + b1_ref[...][None, :])
    _store_split(o_ref, jnp.dot(h, w2_ref[...],
                                preferred_element_type=jnp.float32)
                 + b2_ref[...][None, :])


def _enc(x, p):
    di, dh = p['w1'].shape
    return pl.pallas_call(
        _enc_body, grid=_GRID,
        in_specs=[_row_spec(di), _full_spec(di, dh), _vec_spec(dh),
                  _full_spec(dh, D), _vec_spec(D)],
        out_specs=_split_spec(),
        out_shape=_split_out(),
    )(x, p['w1'], p['b1'], p['w2'], p['b2'])


def _addmm_body(h_ref, a_ref, w_ref, b_ref, o_ref):
    _store_split(o_ref, _join(h_ref) + jnp.dot(
        _join(a_ref), w_ref[...],
        preferred_element_type=jnp.float32) + b_ref[...][None, :])


def _addmm(h, a, w, b):
    return pl.pallas_call(
        _addmm_body, grid=_GRID,
        in_specs=[_split_spec(), _split_spec(), _full_spec(D, D),
                  _vec_spec(D)],
        out_specs=_split_spec(),
        out_shape=_split_out(),
    )(h, a, w, b)


def _ln_act(x, g, b):
    m = jnp.mean(x, axis=-1, keepdims=True)
    v = jnp.mean((x - m) * (x - m), axis=-1, keepdims=True)
    return _leaky((x - m) / jnp.sqrt(v + 1e-5) * g[None, :] + b[None, :])


def _addmm_ln_body(h_ref, a_ref, w_ref, b_ref, g_ref, bn_ref, o_ref):
    t = _join(h_ref) + jnp.dot(_join(a_ref), w_ref[...],
                               preferred_element_type=jnp.float32) + b_ref[...][None, :]
    _store_split(o_ref, _ln_act(t, g_ref[...], bn_ref[...]))


def _addmm_ln(h, a, w, b, g, bn):
    return pl.pallas_call(
        _addmm_ln_body, grid=_GRID,
        in_specs=[_split_spec(), _split_spec(), _full_spec(D, D),
                  _vec_spec(D), _vec_spec(D), _vec_spec(D)],
        out_specs=_split_spec(),
        out_shape=_split_out(),
    )(h, a, w, b, g, bn)


def _lnact_body(x_ref, g_ref, bn_ref, o_ref):
    _store_split(o_ref, _ln_act(_join(x_ref), g_ref[...], bn_ref[...]))


def _lnact(x, g, bn):
    return pl.pallas_call(
        _lnact_body, grid=_GRID,
        in_specs=[_split_spec(), _vec_spec(D), _vec_spec(D)],
        out_specs=_split_spec(),
        out_shape=_split_out(),
    )(x, g, bn)


def _head_body(h_ref, w1_ref, b1_ref, w2_ref, b2_ref, o_ref):
    a = _leaky(jnp.dot(_join(h_ref), w1_ref[...],
                       preferred_element_type=jnp.float32) + b1_ref[...][None, :])
    o_ref[...] = jnp.dot(a, w2_ref[...],
                         preferred_element_type=jnp.float32) + b2_ref[...][None, :]


def _head(h, p1, p2):
    dh = p1['w'].shape[1]
    do = p2['w'].shape[1]
    return pl.pallas_call(
        _head_body, grid=_GRID,
        in_specs=[_split_spec(), _full_spec(D, dh), _vec_spec(dh),
                  _full_spec(dh, do), _vec_spec(do)],
        out_specs=pl.BlockSpec((_BLK, do), lambda i: (i, 0)),
        out_shape=jax.ShapeDtypeStruct((N, do), jnp.float32),
    )(h, p1['w'], p1['b'], p2['w'], p2['b'])


def kernel(x_node, x_net, edge_index_node_to_net, edge_weight_node_to_net,
           edge_type_node_to_net, edge_index_net_to_node,
           edge_weight_net_to_node, params):
    src1 = edge_index_node_to_net[0]
    dst1 = edge_index_node_to_net[1]
    src2 = edge_index_net_to_node[0]
    dst2 = edge_index_net_to_node[1]
    ew1 = edge_weight_node_to_net
    ew2 = edge_weight_net_to_node

    h_inst = _enc(x_node, params['node_enc'])
    h_net = _enc(x_net, params['net_enc'])
    for lp in params['layers']:
        agg_net = segsum(h_inst, src1, dst1, ew1)
        tmp_net = _addmm(h_net, agg_net, lp['w_n2n'], lp['b_n2n'])
        agg_node = segsum(tmp_net, src2, dst2, ew2)
        h_inst = _addmm_ln(h_inst, agg_node, lp['w_net2n'], lp['b_net2n'],
                           lp['ln_g'], lp['ln_b'])
        h_net = _lnact(tmp_net, lp['ln_g'], lp['ln_b'])
    out_inst = _head(h_inst, params['fc1_node'], params['fc2_node'])
    out_net = _head(h_net, params['fc1_net'], params['fc2_net'])
    return out_inst, out_net


# speedup vs baseline: 6.6349x; 1.0380x over previous
import functools

import jax
import jax.numpy as jnp
from jax import lax
from jax.experimental import pallas as pl
from jax.experimental.pallas import tpu as pltpu
from jax.experimental.pallas import tpu_sc as plsc

N = 10000
D = 256
DH = D // 2
E = 320000
NS = 16
NC = 2
ES = E // NS
CHUNK = 4000
NCHUNK = ES // CHUNK
K = 40
NWIN = CHUNK // K
ACC_ROWS = N


def _sc_body(h3, src, dst, w, out3,
             stg_src, stg_dst, stg_w, r0, r1, m0, m1, acc, g0, g1, s0, s1):
    c = lax.axis_index("c")
    s = lax.axis_index("s")
    zf = jnp.zeros((16,), jnp.float32)

    def _zrow(e, _):
        for k in range(DH // 16):
            m0[e, pl.ds(16 * k, 16)] = zf
        return 0
    lax.fori_loop(0, K, _zrow, 0)
    zslab = ACC_ROWS // NS
    for k in range(zslab // K):
        pltpu.sync_copy(m0, acc.at[pl.ds(s * zslab + k * K, K)])
    zrem = zslab - (zslab // K) * K
    pltpu.sync_copy(m0.at[pl.ds(0, zrem)],
                    acc.at[pl.ds(s * zslab + zslab - zrem, zrem)])

    plsc.subcore_barrier()

    bufs = ((r0, m0, g0, s0), (r1, m1, g1, s1))
    ebase = s * ES

    def _gather(wi, ri, gi):
        return pltpu.make_async_copy(
            h3.at[c].at[stg_src.at[pl.ds(wi * K, K)]], ri, gi)

    def _scatter(wi, mi, si):
        return pltpu.make_async_copy(
            mi, acc.at[stg_dst.at[pl.ds(wi * K, K)]], si)

    def _mul(base_e, ri, mi):
        @plsc.parallel_loop(0, K, unroll=8)
        def _edge(e):
            wv = plsc.load_gather(stg_w, [jnp.broadcast_to(base_e + e, (16,))])
            for k in range(DH // 16):
                mi[e, pl.ds(16 * k, 16)] = ri[e, pl.ds(16 * k, 16)] * wv

    def _chunk(ci, _):
        st = ebase + ci * CHUNK
        pltpu.sync_copy(src.at[pl.ds(st, CHUNK)], stg_src)
        pltpu.sync_copy(dst.at[pl.ds(st, CHUNK)], stg_dst)
        pltpu.sync_copy(w.at[pl.ds(st, CHUNK)], stg_w)

        for i in range(2):
            _gather(i, bufs[i][0], bufs[i][2]).start()

        def _pair(j2, _):
            for i in range(2):
                ri, mi, gi, si = bufs[i]
                wi = 2 * j2 + i
                _gather(wi, ri, gi).wait()

                @pl.when(wi >= 2)
                def _():
                    _scatter(wi - 2, mi, si).wait()

                _mul(wi * K, ri, mi)
                pltpu.async_copy(
                    mi, acc.at[stg_dst.at[pl.ds(wi * K, K)]], si, add=True)

                @pl.when(wi + 2 < NWIN)
                def _():
                    _gather(wi + 2, ri, gi).start()
            return 0

        lax.fori_loop(0, NWIN // 2, _pair, 0)
        for i in range(2):
            _scatter(NWIN - 2 + i, bufs[i][1], bufs[i][3]).wait()
        return 0

    lax.fori_loop(0, NCHUNK, _chunk, 0)

    plsc.subcore_barrier()

    wb = N // NS
    for k in range(wb // K):
        pltpu.sync_copy(acc.at[pl.ds(s * wb + k * K, K)],
                        out3.at[c, pl.ds(s * wb + k * K, K)])
    rem = wb - (wb // K) * K
    pltpu.sync_copy(acc.at[pl.ds(s * wb + wb - rem, rem)],
                    out3.at[c, pl.ds(s * wb + wb - rem, rem)])


_segsum = functools.partial(
    pl.kernel,
    out_type=jax.ShapeDtypeStruct((NC, N, DH), jnp.float32),
    mesh=plsc.VectorSubcoreMesh(
        core_axis_name="c", subcore_axis_name="s",
        num_cores=NC, num_subcores=NS),
    compiler_params=pltpu.CompilerParams(
        use_tc_tiling_on_sc=False, needs_layout_passes=False),
    scratch_types=[
        pltpu.VMEM((CHUNK,), jnp.int32),
        pltpu.VMEM((CHUNK,), jnp.int32),
        pltpu.VMEM((CHUNK,), jnp.float32),
        pltpu.VMEM((K, DH), jnp.float32),
        pltpu.VMEM((K, DH), jnp.float32),
        pltpu.VMEM((K, DH), jnp.float32),
        pltpu.VMEM((K, DH), jnp.float32),
        pltpu.VMEM_SHARED((ACC_ROWS, DH), jnp.float32),
        pltpu.SemaphoreType.DMA,
        pltpu.SemaphoreType.DMA,
        pltpu.SemaphoreType.DMA,
        pltpu.SemaphoreType.DMA,
    ],
)(_sc_body)


def segsum(h3, src, dst, w):
    return _segsum(h3, src, dst, w)


_BLK = 2000
_GRID = (N // _BLK,)


def _split_spec():
    return pl.BlockSpec((NC, _BLK, DH), lambda i: (0, i, 0))


def _row_spec(d):
    return pl.BlockSpec((_BLK, d), lambda i: (i, 0))


def _full_spec(a, b):
    return pl.BlockSpec((a, b), lambda i: (0, 0))


def _vec_spec(d):
    return pl.BlockSpec((d,), lambda i: (0,))


def _split_out(d=DH):
    return jax.ShapeDtypeStruct((NC, N, d), jnp.float32)


def _leaky(x):
    return jnp.where(x >= 0, x, 0.01 * x)


def _join(ref):
    return jnp.concatenate([ref[0], ref[1]], axis=-1)


def _store_split(o_ref, y):
    o_ref[0] = y[:, :DH]
    o_ref[1] = y[:, DH:]


def _enc_body(x_ref, w1_ref, b1_ref, w2_ref, b2_ref, o_ref):
    h = _leaky(jnp.dot(x_ref[...], w1_ref[...],
                       preferred_element_type=jnp.float32) + b1_ref[...][None, :])
    _store_split(o_ref, jnp.dot(h, w2_ref[...],
                                preferred_element_type=jnp.float32)
                 + b2_ref[...][None, :])


def _enc(x, p):
    di, dh = p['w1'].shape
    return pl.pallas_call(
        _enc_body, grid=_GRID,
        in_specs=[_row_spec(di), _full_spec(di, dh), _vec_spec(dh),
                  _full_spec(dh, D), _vec_spec(D)],
        out_specs=_split_spec(),
        out_shape=_split_out(),
    )(x, p['w1'], p['b1'], p['w2'], p['b2'])


def _addmm_body(h_ref, a_ref, w_ref, b_ref, o_ref):
    _store_split(o_ref, _join(h_ref) + jnp.dot(
        _join(a_ref), w_ref[...],
        preferred_element_type=jnp.float32) + b_ref[...][None, :])


def _addmm(h, a, w, b):
    return pl.pallas_call(
        _addmm_body, grid=_GRID,
        in_specs=[_split_spec(), _split_spec(), _full_spec(D, D),
                  _vec_spec(D)],
        out_specs=_split_spec(),
        out_shape=_split_out(),
    )(h, a, w, b)


def _ln_act(x, g, b):
    m = jnp.mean(x, axis=-1, keepdims=True)
    v = jnp.mean((x - m) * (x - m), axis=-1, keepdims=True)
    return _leaky((x - m) / jnp.sqrt(v + 1e-5) * g[None, :] + b[None, :])


def _addmm_ln_body(h_ref, a_ref, w_ref, b_ref, g_ref, bn_ref, o_ref):
    t = _join(h_ref) + jnp.dot(_join(a_ref), w_ref[...],
                               preferred_element_type=jnp.float32) + b_ref[...][None, :]
    _store_split(o_ref, _ln_act(t, g_ref[...], bn_ref[...]))


def _addmm_ln(h, a, w, b, g, bn):
    return pl.pallas_call(
        _addmm_ln_body, grid=_GRID,
        in_specs=[_split_spec(), _split_spec(), _full_spec(D, D),
                  _vec_spec(D), _vec_spec(D), _vec_spec(D)],
        out_specs=_split_spec(),
        out_shape=_split_out(),
    )(h, a, w, b, g, bn)


def _lnact_body(x_ref, g_ref, bn_ref, o_ref):
    _store_split(o_ref, _ln_act(_join(x_ref), g_ref[...], bn_ref[...]))


def _lnact(x, g, bn):
    return pl.pallas_call(
        _lnact_body, grid=_GRID,
        in_specs=[_split_spec(), _vec_spec(D), _vec_spec(D)],
        out_specs=_split_spec(),
        out_shape=_split_out(),
    )(x, g, bn)


def _head_body(h_ref, w1_ref, b1_ref, w2_ref, b2_ref, o_ref):
    a = _leaky(jnp.dot(_join(h_ref), w1_ref[...],
                       preferred_element_type=jnp.float32) + b1_ref[...][None, :])
    o_ref[...] = jnp.dot(a, w2_ref[...],
                         preferred_element_type=jnp.float32) + b2_ref[...][None, :]


def _head(h, p1, p2):
    dh = p1['w'].shape[1]
    do = p2['w'].shape[1]
    return pl.pallas_call(
        _head_body, grid=_GRID,
        in_specs=[_split_spec(), _full_spec(D, dh), _vec_spec(dh),
                  _full_spec(dh, do), _vec_spec(do)],
        out_specs=pl.BlockSpec((_BLK, do), lambda i: (i, 0)),
        out_shape=jax.ShapeDtypeStruct((N, do), jnp.float32),
    )(h, p1['w'], p1['b'], p2['w'], p2['b'])


def kernel(x_node, x_net, edge_index_node_to_net, edge_weight_node_to_net,
           edge_type_node_to_net, edge_index_net_to_node,
           edge_weight_net_to_node, params):
    src1 = edge_index_node_to_net[0]
    dst1 = edge_index_node_to_net[1]
    src2 = edge_index_net_to_node[0]
    dst2 = edge_index_net_to_node[1]
    ew1 = edge_weight_node_to_net
    ew2 = edge_weight_net_to_node

    h_inst = _enc(x_node, params['node_enc'])
    h_net = _enc(x_net, params['net_enc'])
    for lp in params['layers']:
        agg_net = segsum(h_inst, src1, dst1, ew1)
        tmp_net = _addmm(h_net, agg_net, lp['w_n2n'], lp['b_n2n'])
        agg_node = segsum(tmp_net, src2, dst2, ew2)
        h_inst = _addmm_ln(h_inst, agg_node, lp['w_net2n'], lp['b_net2n'],
                           lp['ln_g'], lp['ln_b'])
        h_net = _lnact(tmp_net, lp['ln_g'], lp['ln_b'])
    out_inst = _head(h_inst, params['fc1_node'], params['fc2_node'])
    out_net = _head(h_net, params['fc1_net'], params['fc2_net'])
    return out_inst, out_net


# speedup vs baseline: 8.2818x; 1.2482x over previous
import functools

import jax
import jax.numpy as jnp
from jax import lax
from jax.experimental import pallas as pl
from jax.experimental.pallas import tpu as pltpu
from jax.experimental.pallas import tpu_sc as plsc

N = 10000
D = 256
DH = D // 2
E = 320000
NS = 16
NC = 2
ES = E // NS
CHUNK = 4000
NCHUNK = ES // CHUNK
K = 40
NWIN = CHUNK // K
ACC_ROWS = N


def _sc_body(h3, src, dst, w, out3,
             stg_src, stg_dst, stg_w, r0, r1, r2, r3, m0, m1, acc,
             g0, g1, g2, g3, s0, s1):
    c = lax.axis_index("c")
    s = lax.axis_index("s")
    zf = jnp.zeros((16,), jnp.float32)

    def _zrow(e, _):
        for k in range(DH // 16):
            m0[e, pl.ds(16 * k, 16)] = zf
        return 0
    lax.fori_loop(0, K, _zrow, 0)
    zslab = ACC_ROWS // NS
    for k in range(zslab // K):
        pltpu.sync_copy(m0, acc.at[pl.ds(s * zslab + k * K, K)])
    zrem = zslab - (zslab // K) * K
    pltpu.sync_copy(m0.at[pl.ds(0, zrem)],
                    acc.at[pl.ds(s * zslab + zslab - zrem, zrem)])

    plsc.subcore_barrier()

    gbufs = ((r0, g0), (r1, g1), (r2, g2), (r3, g3))
    mbufs = ((m0, s0), (m1, s1))
    ebase = s * ES

    def _gather(wi, ri, gi):
        return pltpu.make_async_copy(
            h3.at[c].at[stg_src.at[pl.ds(wi * K, K)]], ri, gi)

    def _scatter(wi, mi, si):
        return pltpu.make_async_copy(
            mi, acc.at[stg_dst.at[pl.ds(wi * K, K)]], si)

    def _mul(base_e, ri, mi):
        @plsc.parallel_loop(0, K, unroll=8)
        def _edge(e):
            wv = plsc.load_gather(stg_w, [jnp.broadcast_to(base_e + e, (16,))])
            for k in range(DH // 16):
                mi[e, pl.ds(16 * k, 16)] = ri[e, pl.ds(16 * k, 16)] * wv

    def _chunk(ci, _):
        st = ebase + ci * CHUNK
        pltpu.sync_copy(src.at[pl.ds(st, CHUNK)], stg_src)
        pltpu.sync_copy(dst.at[pl.ds(st, CHUNK)], stg_dst)
        pltpu.sync_copy(w.at[pl.ds(st, CHUNK)], stg_w)

        for i in range(4):
            _gather(i, gbufs[i][0], gbufs[i][1]).start()

        def _quad(j4, _):
            for i in range(4):
                ri, gi = gbufs[i]
                mi, si = mbufs[i % 2]
                wi = 4 * j4 + i
                _gather(wi, ri, gi).wait()

                @pl.when(wi >= 2)
                def _():
                    _scatter(wi - 2, mi, si).wait()

                _mul(wi * K, ri, mi)
                pltpu.async_copy(
                    mi, acc.at[stg_dst.at[pl.ds(wi * K, K)]], si, add=True)

                @pl.when(wi + 4 < NWIN)
                def _():
                    _gather(wi + 4, ri, gi).start()
            return 0

        lax.fori_loop(0, NWIN // 4, _quad, 0)
        for i in range(2):
            _scatter(NWIN - 2 + i, mbufs[i][0], mbufs[i][1]).wait()
        return 0

    lax.fori_loop(0, NCHUNK, _chunk, 0)

    plsc.subcore_barrier()

    wb = N // NS
    for k in range(wb // K):
        pltpu.sync_copy(acc.at[pl.ds(s * wb + k * K, K)],
                        out3.at[c, pl.ds(s * wb + k * K, K)])
    rem = wb - (wb // K) * K
    pltpu.sync_copy(acc.at[pl.ds(s * wb + wb - rem, rem)],
                    out3.at[c, pl.ds(s * wb + wb - rem, rem)])


_segsum = functools.partial(
    pl.kernel,
    out_type=jax.ShapeDtypeStruct((NC, N, DH), jnp.float32),
    mesh=plsc.VectorSubcoreMesh(
        core_axis_name="c", subcore_axis_name="s",
        num_cores=NC, num_subcores=NS),
    compiler_params=pltpu.CompilerParams(
        use_tc_tiling_on_sc=False, needs_layout_passes=False),
    scratch_types=[
        pltpu.VMEM((CHUNK,), jnp.int32),
        pltpu.VMEM((CHUNK,), jnp.int32),
        pltpu.VMEM((CHUNK,), jnp.float32),
        pltpu.VMEM((K, DH), jnp.float32),
        pltpu.VMEM((K, DH), jnp.float32),
        pltpu.VMEM((K, DH), jnp.float32),
        pltpu.VMEM((K, DH), jnp.float32),
        pltpu.VMEM((K, DH), jnp.float32),
        pltpu.VMEM((K, DH), jnp.float32),
        pltpu.VMEM_SHARED((ACC_ROWS, DH), jnp.float32),
        pltpu.SemaphoreType.DMA,
        pltpu.SemaphoreType.DMA,
        pltpu.SemaphoreType.DMA,
        pltpu.SemaphoreType.DMA,
        pltpu.SemaphoreType.DMA,
        pltpu.SemaphoreType.DMA,
    ],
)(_sc_body)


def segsum(h3, src, dst, w):
    return _segsum(h3, src, dst, w)


_BLK = 2000
_GRID = (N // _BLK,)


def _split_spec():
    return pl.BlockSpec((NC, _BLK, DH), lambda i: (0, i, 0))


def _row_spec(d):
    return pl.BlockSpec((_BLK, d), lambda i: (i, 0))


def _full_spec(a, b):
    return pl.BlockSpec((a, b), lambda i: (0, 0))


def _vec_spec(d):
    return pl.BlockSpec((d,), lambda i: (0,))


def _split_out(d=DH):
    return jax.ShapeDtypeStruct((NC, N, d), jnp.float32)


def _leaky(x):
    return jnp.where(x >= 0, x, 0.01 * x)


def _join(ref):
    return jnp.concatenate([ref[0], ref[1]], axis=-1)


def _store_split(o_ref, y):
    o_ref[0] = y[:, :DH]
    o_ref[1] = y[:, DH:]


def _enc_body(x_ref, w1_ref, b1_ref, w2_ref, b2_ref, o_ref):
    h = _leaky(jnp.dot(x_ref[...], w1_ref[...],
                       preferred_element_type=jnp.float32) + b1_ref[...][None, :])
    _store_split(o_ref, jnp.dot(h, w2_ref[...],
                                preferred_element_type=jnp.float32)
                 + b2_ref[...][None, :])


def _enc(x, p):
    di, dh = p['w1'].shape
    return pl.pallas_call(
        _enc_body, grid=_GRID,
        in_specs=[_row_spec(di), _full_spec(di, dh), _vec_spec(dh),
                  _full_spec(dh, D), _vec_spec(D)],
        out_specs=_split_spec(),
        out_shape=_split_out(),
    )(x, p['w1'], p['b1'], p['w2'], p['b2'])


def _addmm_body(h_ref, a_ref, w_ref, b_ref, o_ref):
    _store_split(o_ref, _join(h_ref) + jnp.dot(
        _join(a_ref), w_ref[...],
        preferred_element_type=jnp.float32) + b_ref[...][None, :])


def _addmm(h, a, w, b):
    return pl.pallas_call(
        _addmm_body, grid=_GRID,
        in_specs=[_split_spec(), _split_spec(), _full_spec(D, D),
                  _vec_spec(D)],
        out_specs=_split_spec(),
        out_shape=_split_out(),
    )(h, a, w, b)


def _ln_act(x, g, b):
    m = jnp.mean(x, axis=-1, keepdims=True)
    v = jnp.mean((x - m) * (x - m), axis=-1, keepdims=True)
    return _leaky((x - m) / jnp.sqrt(v + 1e-5) * g[None, :] + b[None, :])


def _addmm_ln_body(h_ref, a_ref, w_ref, b_ref, g_ref, bn_ref, o_ref):
    t = _join(h_ref) + jnp.dot(_join(a_ref), w_ref[...],
                               preferred_element_type=jnp.float32) + b_ref[...][None, :]
    _store_split(o_ref, _ln_act(t, g_ref[...], bn_ref[...]))


def _addmm_ln(h, a, w, b, g, bn):
    return pl.pallas_call(
        _addmm_ln_body, grid=_GRID,
        in_specs=[_split_spec(), _split_spec(), _full_spec(D, D),
                  _vec_spec(D), _vec_spec(D), _vec_spec(D)],
        out_specs=_split_spec(),
        out_shape=_split_out(),
    )(h, a, w, b, g, bn)


def _lnact_body(x_ref, g_ref, bn_ref, o_ref):
    _store_split(o_ref, _ln_act(_join(x_ref), g_ref[...], bn_ref[...]))


def _lnact(x, g, bn):
    return pl.pallas_call(
        _lnact_body, grid=_GRID,
        in_specs=[_split_spec(), _vec_spec(D), _vec_spec(D)],
        out_specs=_split_spec(),
        out_shape=_split_out(),
    )(x, g, bn)


def _head_body(h_ref, w1_ref, b1_ref, w2_ref, b2_ref, o_ref):
    a = _leaky(jnp.dot(_join(h_ref), w1_ref[...],
                       preferred_element_type=jnp.float32) + b1_ref[...][None, :])
    o_ref[...] = jnp.dot(a, w2_ref[...],
                         preferred_element_type=jnp.float32) + b2_ref[...][None, :]


def _head(h, p1, p2):
    dh = p1['w'].shape[1]
    do = p2['w'].shape[1]
    return pl.pallas_call(
        _head_body, grid=_GRID,
        in_specs=[_split_spec(), _full_spec(D, dh), _vec_spec(dh),
                  _full_spec(dh, do), _vec_spec(do)],
        out_specs=pl.BlockSpec((_BLK, do), lambda i: (i, 0)),
        out_shape=jax.ShapeDtypeStruct((N, do), jnp.float32),
    )(h, p1['w'], p1['b'], p2['w'], p2['b'])


def kernel(x_node, x_net, edge_index_node_to_net, edge_weight_node_to_net,
           edge_type_node_to_net, edge_index_net_to_node,
           edge_weight_net_to_node, params):
    src1 = edge_index_node_to_net[0]
    dst1 = edge_index_node_to_net[1]
    src2 = edge_index_net_to_node[0]
    dst2 = edge_index_net_to_node[1]
    ew1 = edge_weight_node_to_net
    ew2 = edge_weight_net_to_node

    h_inst = _enc(x_node, params['node_enc'])
    h_net = _enc(x_net, params['net_enc'])
    for lp in params['layers']:
        agg_net = segsum(h_inst, src1, dst1, ew1)
        tmp_net = _addmm(h_net, agg_net, lp['w_n2n'], lp['b_n2n'])
        agg_node = segsum(tmp_net, src2, dst2, ew2)
        h_inst = _addmm_ln(h_inst, agg_node, lp['w_net2n'], lp['b_net2n'],
                           lp['ln_g'], lp['ln_b'])
        h_net = _lnact(tmp_net, lp['ln_g'], lp['ln_b'])
    out_inst = _head(h_inst, params['fc1_node'], params['fc2_node'])
    out_net = _head(h_net, params['fc1_net'], params['fc2_net'])
    return out_inst, out_net
